# baseline (speedup 1.0000x reference)
import jax
import jax.numpy as jnp
from jax.experimental import pallas as pl
from jax.experimental.pallas import tpu as pltpu

_NUM_Q = 8
_K = 1024
_COMMIT = 0.25
_TILE = 2048
_NSPLIT = 2
_CHUNK = _TILE // _NSPLIT


def _rvq_body(x_ref, xlast_ref, cb_ref,
              quant_ref, idx_ref, sse_ref, resid_ref, esq_ref,
              hi_ref, mid_ref, lo_ref):
    q = pl.program_id(0)
    i = pl.program_id(1)

    @pl.when(jnp.logical_and(q == 0, i == 0))
    def _init_sse():
        sse_ref[...] = jnp.zeros_like(sse_ref)

    @pl.when(q == 0)
    def _init_resid():
        resid_ref[i] = x_ref[...]

    r = resid_ref[i]
    emb = cb_ref[0]

    @pl.when(i == 0)
    def _init_esq():
        esq_ref[...] = jnp.sum(emb * emb, axis=1)[None, :]
        hi = emb.astype(jnp.bfloat16)
        hi32 = hi.astype(jnp.float32)
        mid = (emb - hi32).astype(jnp.bfloat16)
        mid32 = mid.astype(jnp.float32)
        hi_ref[...] = hi
        mid_ref[...] = mid
        lo_ref[...] = ((emb - hi32) - mid32).astype(jnp.bfloat16)

    emb2 = emb + emb
    def _half(rh):
        m2 = jax.lax.dot_general(
            rh, emb2, (((1,), (1,)), ((), ())),
            precision=jax.lax.Precision.DEFAULT,
            preferred_element_type=jnp.float32)
        xsq = jnp.sum(rh * rh, axis=1, keepdims=True)
        esq = esq_ref[...]
        dist = (xsq + esq) - m2

        dmin = jnp.min(dist, axis=1, keepdims=True)
        iota = jax.lax.broadcasted_iota(jnp.int32, (_CHUNK, _K), 1)
        idx = jnp.min(jnp.where(dist == dmin, iota, _K), axis=1)
        onehot = (iota == idx[:, None]).astype(jnp.bfloat16)
        qv = (jax.lax.dot_general(
                  onehot, hi_ref[...], (((1,), (0,)), ((), ())),
                  preferred_element_type=jnp.float32)
              + jax.lax.dot_general(
                  onehot, mid_ref[...], (((1,), (0,)), ((), ())),
                  preferred_element_type=jnp.float32)
              + jax.lax.dot_general(
                  onehot, lo_ref[...], (((1,), (0,)), ((), ())),
                  preferred_element_type=jnp.float32))

        diff = qv - rh
        q_st = rh + diff
        r_next = rh - q_st
        return idx, r_next, jnp.sum(dmin)

    parts = [_half(r[j * _CHUNK:(j + 1) * _CHUNK]) for j in range(_NSPLIT)]
    r_next = jnp.concatenate([p[1] for p in parts], axis=0)
    idx = jnp.concatenate([p[0] for p in parts], axis=0)

    s_all = parts[0][2]
    for p in parts[1:]:
        s_all = s_all + p[2]
    sse_ref[...] = sse_ref[...] + s_all.reshape(1, 1)
    resid_ref[i] = r_next
    idx_ref[...] = idx.reshape(1, 1, 1, _TILE)

    @pl.when(q == _NUM_Q - 1)
    def _emit_quant():
        quant_ref[...] = xlast_ref[...] - r_next


def kernel(x, codebooks):
    B, D, T = x.shape
    tokens = B * T
    ntiles = tokens // _TILE
    x_flat = jnp.transpose(x, (0, 2, 1)).reshape(tokens, D)

    quant_flat, idx_out, sse = pl.pallas_call(
        _rvq_body,
        grid=(_NUM_Q, ntiles),
        in_specs=[
            pl.BlockSpec((_TILE, D), lambda q, i: (jnp.where(q == 0, i, 0), 0)),
            pl.BlockSpec((_TILE, D),
                         lambda q, i: (jnp.where(q == _NUM_Q - 1, i, 0), 0)),
            pl.BlockSpec((1, _K, D), lambda q, i: (q, 0, 0)),
        ],
        out_specs=[
            pl.BlockSpec((_TILE, D),
                         lambda q, i: (jnp.where(q == _NUM_Q - 1, i, 0), 0)),
            pl.BlockSpec((1, 1, 1, _TILE), lambda q, i: (q, i, 0, 0)),
            pl.BlockSpec((1, 1), lambda q, i: (0, 0)),
        ],
        out_shape=[
            jax.ShapeDtypeStruct((tokens, D), jnp.float32),
            jax.ShapeDtypeStruct((_NUM_Q, ntiles, 1, _TILE), jnp.int32),
            jax.ShapeDtypeStruct((1, 1), jnp.float32),
        ],
        scratch_shapes=[pltpu.VMEM((ntiles, _TILE, D), jnp.float32),
                        pltpu.VMEM((1, _K), jnp.float32),
                        pltpu.VMEM((_K, D), jnp.bfloat16),
                        pltpu.VMEM((_K, D), jnp.bfloat16),
                        pltpu.VMEM((_K, D), jnp.bfloat16)],
        compiler_params=pltpu.CompilerParams(
            dimension_semantics=("arbitrary", "parallel")),
    )(x_flat, x_flat, codebooks)

    quantized = jnp.transpose(quant_flat.reshape(B, T, D), (0, 2, 1))
    indices = jnp.transpose(idx_out.reshape(_NUM_Q, B, T), (1, 0, 2))
    total_loss = sse[0, 0] * (_COMMIT / (B * D * T))
    return quantized, indices, total_loss


# speedup vs baseline: 1.0039x; 1.0039x over previous
import jax
import jax.numpy as jnp
from jax.experimental import pallas as pl
from jax.experimental.pallas import tpu as pltpu

_NUM_Q = 8
_K = 1024
_COMMIT = 0.25
_TILE = 2048
_NSPLIT = 2
_CHUNK = _TILE // _NSPLIT


def _rvq_body(x_ref, xlast_ref, cb_ref,
              quant_ref, idx_ref, sse_ref, resid_ref, esq_ref,
              hi_ref, mid_ref, lo_ref):
    q = pl.program_id(0)
    i = pl.program_id(1)

    @pl.when(jnp.logical_and(q == 0, i == 0))
    def _init_sse():
        sse_ref[...] = jnp.zeros_like(sse_ref)

    @pl.when(q == 0)
    def _init_resid():
        resid_ref[i] = x_ref[...]

    r = resid_ref[i]
    emb = cb_ref[0]

    @pl.when(i == 0)
    def _init_esq():
        esq_ref[...] = jnp.sum(emb * emb, axis=1)[None, :]
        hi = emb.astype(jnp.bfloat16)
        hi32 = hi.astype(jnp.float32)
        mid = (emb - hi32).astype(jnp.bfloat16)
        mid32 = mid.astype(jnp.float32)
        hi_ref[...] = hi
        mid_ref[...] = mid
        lo_ref[...] = ((emb - hi32) - mid32).astype(jnp.bfloat16)

    emb2 = emb + emb
    def _half(rh):
        m2 = jax.lax.dot_general(
            rh, emb2, (((1,), (1,)), ((), ())),
            precision=jax.lax.Precision.DEFAULT,
            preferred_element_type=jnp.float32)
        xsq = jnp.sum(rh * rh, axis=1, keepdims=True)
        esq = esq_ref[...]
        dist = (xsq + esq) - m2

        dmin = jnp.min(dist, axis=1, keepdims=True)
        iota = jax.lax.broadcasted_iota(jnp.int32, (_CHUNK, _K), 1)
        idx = jnp.min(jnp.where(dist == dmin, iota, _K), axis=1)
        onehot = (iota == idx[:, None]).astype(jnp.bfloat16)
        qv = (jax.lax.dot_general(
                  onehot, hi_ref[...], (((1,), (0,)), ((), ())),
                  preferred_element_type=jnp.float32)
              + jax.lax.dot_general(
                  onehot, mid_ref[...], (((1,), (0,)), ((), ())),
                  preferred_element_type=jnp.float32)
              + jax.lax.dot_general(
                  onehot, lo_ref[...], (((1,), (0,)), ((), ())),
                  preferred_element_type=jnp.float32))

        diff = qv - rh
        q_st = rh + diff
        r_next = rh - q_st
        return idx, r_next, jnp.sum(dmin)

    parts = [_half(r[j * _CHUNK:(j + 1) * _CHUNK]) for j in range(_NSPLIT)]
    r_next = jnp.concatenate([p[1] for p in parts], axis=0)
    idx = jnp.concatenate([p[0] for p in parts], axis=0)

    s_all = parts[0][2]
    for p in parts[1:]:
        s_all = s_all + p[2]
    sse_ref[...] = sse_ref[...] + s_all.reshape(1, 1)
    resid_ref[i] = r_next
    idx_ref[...] = idx.reshape(1, 1, 1, _TILE)

    @pl.when(q == _NUM_Q - 1)
    def _emit_quant():
        quant_ref[...] = xlast_ref[...] - r_next


def kernel(x, codebooks):
    B, D, T = x.shape
    tokens = B * T
    ntiles = tokens // _TILE
    x_flat = jnp.transpose(x, (0, 2, 1)).reshape(tokens, D)

    quant_flat, idx_out, sse = pl.pallas_call(
        _rvq_body,
        grid=(_NUM_Q, ntiles),
        in_specs=[
            pl.BlockSpec((_TILE, D), lambda q, i: (jnp.where(q == 0, i, 0), 0)),
            pl.BlockSpec((_TILE, D),
                         lambda q, i: (jnp.where(q == _NUM_Q - 1, i, 0), 0)),
            pl.BlockSpec((1, _K, D), lambda q, i: (q, 0, 0)),
        ],
        out_specs=[
            pl.BlockSpec((_TILE, D),
                         lambda q, i: (jnp.where(q == _NUM_Q - 1, i, 0), 0)),
            pl.BlockSpec((1, 1, 1, _TILE), lambda q, i: (q, i, 0, 0)),
            pl.BlockSpec((1, 1), lambda q, i: (0, 0)),
        ],
        out_shape=[
            jax.ShapeDtypeStruct((tokens, D), jnp.float32),
            jax.ShapeDtypeStruct((_NUM_Q, ntiles, 1, _TILE), jnp.int32),
            jax.ShapeDtypeStruct((1, 1), jnp.float32),
        ],
        scratch_shapes=[pltpu.VMEM((ntiles, _TILE, D), jnp.float32),
                        pltpu.VMEM((1, _K), jnp.float32),
                        pltpu.VMEM((_K, D), jnp.bfloat16),
                        pltpu.VMEM((_K, D), jnp.bfloat16),
                        pltpu.VMEM((_K, D), jnp.bfloat16)],
    )(x_flat, x_flat, codebooks)

    quantized = jnp.transpose(quant_flat.reshape(B, T, D), (0, 2, 1))
    indices = jnp.transpose(idx_out.reshape(_NUM_Q, B, T), (1, 0, 2))
    total_loss = sse[0, 0] * (_COMMIT / (B * D * T))
    return quantized, indices, total_loss
